# Initial kernel scaffold; baseline (speedup 1.0000x reference)
import functools

import jax
import jax.numpy as jnp
from jax import lax
from jax.experimental import pallas as pl
from jax.experimental.pallas import tpu as pltpu
from jax.experimental.pallas import tpu_sc as plsc

N_NODES = 100000
N_EDGES = 6400000
EPS = 1e-06

NW = 32
PAD_N = 102400
NODE_CHUNK = PAD_N // NW
EDGES_PER_TILE = N_EDGES // NW
EDGE_CHUNK = 4000
N_EDGE_CHUNKS = EDGES_PER_TILE // EDGE_CHUNK
L = 16


def _sc_body(d_hbm, ei_hbm, mv_hbm, r_hbm, gt_hbm, out_hbm,
             d_tab, src_v, dst_v, a_v, r_v, gt_v, stage):
    cid = lax.axis_index("c")
    sid = lax.axis_index("s")
    wid = sid * 2 + cid

    pltpu.sync_copy(d_hbm, d_tab)

    zero = jnp.zeros((L,), jnp.float32)

    nbase = wid * NODE_CHUNK
    pltpu.sync_copy(r_hbm.at[pl.ds(nbase, NODE_CHUNK)], r_v)
    pltpu.sync_copy(gt_hbm.at[pl.ds(nbase, NODE_CHUNK)], gt_v)

    def node_step(i, accs):
        a2, a3, a4, a5 = accs
        dv = d_tab[pl.ds(nbase + i * L, L)]
        rv = r_v[pl.ds(i * L, L)]
        gv = gt_v[pl.ds(i * L, L)]
        return (a2 + rv * dv, a3 + dv * dv, a4 + dv * gv, a5 + gv * gv)

    a2, a3, a4, a5 = lax.fori_loop(0, NODE_CHUNK // L, node_step,
                                   (zero, zero, zero, zero))

    ebase = wid * EDGES_PER_TILE

    def chunk_step(c, acc):
        base = ebase + c * EDGE_CHUNK
        pltpu.sync_copy(ei_hbm.at[pl.ds(base, EDGE_CHUNK)], src_v)
        pltpu.sync_copy(ei_hbm.at[pl.ds(N_EDGES + base, EDGE_CHUNK)], dst_v)
        pltpu.sync_copy(mv_hbm.at[pl.ds(base, EDGE_CHUNK)], a_v)

        def edge_step(j, acc_in):
            si = src_v[pl.ds(j * L, L)]
            ti = dst_v[pl.ds(j * L, L)]
            av = a_v[pl.ds(j * L, L)]
            dsv = plsc.load_gather(d_tab, [si])
            dtv = plsc.load_gather(d_tab, [ti])
            return acc_in + av * dsv * dtv

        return lax.fori_loop(0, EDGE_CHUNK // L, edge_step, acc)

    a1 = lax.fori_loop(0, N_EDGE_CHUNKS, chunk_step, zero)

    stage[pl.ds(0, L)] = a1
    stage[pl.ds(L, L)] = a2
    stage[pl.ds(2 * L, L)] = a3
    stage[pl.ds(3 * L, L)] = a4
    stage[pl.ds(4 * L, L)] = a5
    pltpu.sync_copy(stage, out_hbm.at[wid])


@jax.jit
def _run(d_pad, ei_flat, mv, r_pad, gt_pad):
    mesh = plsc.VectorSubcoreMesh(core_axis_name="c", subcore_axis_name="s")
    f = pl.kernel(
        _sc_body,
        out_type=jax.ShapeDtypeStruct((NW, 5 * L), jnp.float32),
        mesh=mesh,
        scratch_types=[
            pltpu.VMEM((PAD_N,), jnp.float32),
            pltpu.VMEM((EDGE_CHUNK,), jnp.int32),
            pltpu.VMEM((EDGE_CHUNK,), jnp.int32),
            pltpu.VMEM((EDGE_CHUNK,), jnp.float32),
            pltpu.VMEM((NODE_CHUNK,), jnp.float32),
            pltpu.VMEM((NODE_CHUNK,), jnp.float32),
            pltpu.VMEM((5 * L,), jnp.float32),
        ],
    )
    partials = f(d_pad, ei_flat, mv, r_pad, gt_pad)
    s = jnp.sum(partials.reshape(NW, 5, L), axis=(0, 2))
    alpha = s[1] / (s[0] + EPS)
    return (alpha * alpha * s[2] - 2.0 * alpha * s[3] + s[4]) / (s[4] + EPS)


def kernel(d, residual, gt, edge_index, matrix_values, mask, L_values,
           batch_ids):
    del mask, L_values, batch_ids
    pad = PAD_N - N_NODES
    d_pad = jnp.pad(d, (0, pad))
    r_pad = jnp.pad(residual, (0, pad))
    gt_pad = jnp.pad(gt, (0, pad))
    ei_flat = edge_index.astype(jnp.int32).reshape(-1)
    return _run(d_pad, ei_flat, matrix_values, r_pad, gt_pad)


# speedup vs baseline: 258.7603x; 258.7603x over previous
import functools

import jax
import jax.numpy as jnp
from jax import lax
from jax.experimental import pallas as pl
from jax.experimental.pallas import tpu as pltpu
from jax.experimental.pallas import tpu_sc as plsc

N_NODES = 100000
N_EDGES = 6400000
EPS = 1e-06

NW = 32
PAD_N = 102400
NODE_CHUNK = PAD_N // NW
EDGES_PER_TILE = N_EDGES // NW
EDGE_CHUNK = 4000
N_EDGE_CHUNKS = EDGES_PER_TILE // EDGE_CHUNK
L = 16


def _sc_body(d_hbm, ei_hbm, mv_hbm, r_hbm, gt_hbm, out_hbm,
             d_tab, src_v, dst_v, a_v, r_v, gt_v, stage):
    cid = lax.axis_index("c")
    sid = lax.axis_index("s")
    wid = sid * 2 + cid

    pltpu.sync_copy(d_hbm, d_tab)

    zero = jnp.zeros((L,), jnp.float32)

    nbase = wid * NODE_CHUNK
    pltpu.sync_copy(r_hbm.at[pl.ds(nbase, NODE_CHUNK)], r_v)
    pltpu.sync_copy(gt_hbm.at[pl.ds(nbase, NODE_CHUNK)], gt_v)

    def node_step(i, accs):
        a2, a3, a4, a5 = accs
        dv = d_tab[pl.ds(nbase + i * L, L)]
        rv = r_v[pl.ds(i * L, L)]
        gv = gt_v[pl.ds(i * L, L)]
        return (a2 + rv * dv, a3 + dv * dv, a4 + dv * gv, a5 + gv * gv)

    a2, a3, a4, a5 = lax.fori_loop(0, NODE_CHUNK // L, node_step,
                                   (zero, zero, zero, zero))

    ebase = wid * EDGES_PER_TILE

    def chunk_step(c, acc):
        base = ebase + c * EDGE_CHUNK
        pltpu.sync_copy(ei_hbm.at[pl.ds(base, EDGE_CHUNK)], src_v)
        pltpu.sync_copy(ei_hbm.at[pl.ds(N_EDGES + base, EDGE_CHUNK)], dst_v)
        pltpu.sync_copy(mv_hbm.at[pl.ds(base, EDGE_CHUNK)], a_v)

        def edge_step(j, acc_in):
            si = src_v[pl.ds(j * L, L)]
            ti = dst_v[pl.ds(j * L, L)]
            av = a_v[pl.ds(j * L, L)]
            dsv = plsc.load_gather(d_tab, [si])
            dtv = plsc.load_gather(d_tab, [ti])
            return acc_in + av * dsv * dtv

        return lax.fori_loop(0, EDGE_CHUNK // L, edge_step, acc)

    a1 = lax.fori_loop(0, N_EDGE_CHUNKS, chunk_step, zero)

    stage[pl.ds(0, L)] = a1
    stage[pl.ds(L, L)] = a2
    stage[pl.ds(2 * L, L)] = a3
    stage[pl.ds(3 * L, L)] = a4
    stage[pl.ds(4 * L, L)] = a5
    pltpu.sync_copy(stage, out_hbm.at[wid])


@jax.jit
def _run(d_pad, ei_flat, mv, r_pad, gt_pad):
    mesh = plsc.VectorSubcoreMesh(core_axis_name="c", subcore_axis_name="s")
    f = pl.kernel(
        _sc_body,
        out_type=jax.ShapeDtypeStruct((NW, 5 * L), jnp.float32),
        mesh=mesh,
        scratch_types=[
            pltpu.VMEM((PAD_N,), jnp.float32),
            pltpu.VMEM((EDGE_CHUNK,), jnp.int32),
            pltpu.VMEM((EDGE_CHUNK,), jnp.int32),
            pltpu.VMEM((EDGE_CHUNK,), jnp.float32),
            pltpu.VMEM((NODE_CHUNK,), jnp.float32),
            pltpu.VMEM((NODE_CHUNK,), jnp.float32),
            pltpu.VMEM((5 * L,), jnp.float32),
        ],
        compiler_params=pltpu.CompilerParams(needs_layout_passes=False),
    )
    partials = f(d_pad, ei_flat, mv, r_pad, gt_pad)
    s = jnp.sum(partials.reshape(NW, 5, L), axis=(0, 2))
    alpha = s[1] / (s[0] + EPS)
    return (alpha * alpha * s[2] - 2.0 * alpha * s[3] + s[4]) / (s[4] + EPS)


def kernel(d, residual, gt, edge_index, matrix_values, mask, L_values,
           batch_ids):
    del mask, L_values, batch_ids
    pad = PAD_N - N_NODES
    d_pad = jnp.pad(d, (0, pad))
    r_pad = jnp.pad(residual, (0, pad))
    gt_pad = jnp.pad(gt, (0, pad))
    ei_flat = edge_index.astype(jnp.int32).reshape(-1)
    return _run(d_pad, ei_flat, matrix_values, r_pad, gt_pad)


# speedup vs baseline: 533.8835x; 2.0632x over previous
import jax
import jax.numpy as jnp
from jax import lax
from jax.experimental import pallas as pl
from jax.experimental.pallas import tpu as pltpu
from jax.experimental.pallas import tpu_sc as plsc

N_NODES = 100000
N_EDGES = 6400000
EPS = 1e-06

NW = 32
PAD_N = 102400
NODE_CHUNK = PAD_N // NW
EDGES_PER_TILE = N_EDGES // NW
CHUNK = 4000
N_CHUNKS = EDGES_PER_TILE // CHUNK
L = 16
UNROLL = 5


def _sc_body(d_hbm, ei_hbm, mv_hbm, r_hbm, gt_hbm, out_hbm,
             d_tab, src_a, dst_a, av_a, src_b, dst_b, av_b, stage,
             sem_a, sem_b):
    cid = lax.axis_index("c")
    sid = lax.axis_index("s")
    wid = sid * 2 + cid

    pltpu.sync_copy(d_hbm, d_tab)

    zero = jnp.zeros((L,), jnp.float32)

    nbase = wid * NODE_CHUNK
    pltpu.sync_copy(r_hbm.at[pl.ds(nbase, NODE_CHUNK)],
                    av_a.at[pl.ds(0, NODE_CHUNK)])
    pltpu.sync_copy(gt_hbm.at[pl.ds(nbase, NODE_CHUNK)],
                    av_b.at[pl.ds(0, NODE_CHUNK)])

    def node_step(i, accs):
        a2, a3, a4, a5 = accs
        for k in range(4):
            off = i * 4 * L + k * L
            dv = d_tab[pl.ds(nbase + off, L)]
            rv = av_a[pl.ds(off, L)]
            gv = av_b[pl.ds(off, L)]
            a2 = a2 + rv * dv
            a3 = a3 + dv * dv
            a4 = a4 + dv * gv
            a5 = a5 + gv * gv
        return (a2, a3, a4, a5)

    a2, a3, a4, a5 = lax.fori_loop(0, NODE_CHUNK // (4 * L), node_step,
                                   (zero, zero, zero, zero))

    ebase = wid * EDGES_PER_TILE

    def start(c, sbuf, dbuf, abuf, sem):
        base = ebase + c * CHUNK
        pltpu.async_copy(ei_hbm.at[pl.ds(base, CHUNK)], sbuf, sem)
        pltpu.async_copy(ei_hbm.at[pl.ds(N_EDGES + base, CHUNK)], dbuf, sem)
        pltpu.async_copy(mv_hbm.at[pl.ds(base, CHUNK)], abuf, sem)

    def drain(sbuf, dbuf, abuf, sem):
        dummy = ei_hbm.at[pl.ds(0, CHUNK)]
        pltpu.make_async_copy(dummy, sbuf, sem).wait()
        pltpu.make_async_copy(dummy, dbuf, sem).wait()
        pltpu.make_async_copy(dummy, abuf, sem).wait()

    def compute(sbuf, dbuf, abuf, accs):
        def step(j, accs_in):
            out = []
            for k in range(UNROLL):
                off = j * UNROLL * L + k * L
                si = sbuf[pl.ds(off, L)]
                ti = dbuf[pl.ds(off, L)]
                av = abuf[pl.ds(off, L)]
                dsv = plsc.load_gather(d_tab, [si])
                dtv = plsc.load_gather(d_tab, [ti])
                out.append(accs_in[k] + av * dsv * dtv)
            return tuple(out)
        return lax.fori_loop(0, CHUNK // (UNROLL * L), step, accs)

    start(0, src_a, dst_a, av_a, sem_a)
    start(1, src_b, dst_b, av_b, sem_b)

    def outer(i, accs):
        drain(src_a, dst_a, av_a, sem_a)
        accs = compute(src_a, dst_a, av_a, accs)

        @pl.when(2 * i + 2 < N_CHUNKS)
        def _():
            start(2 * i + 2, src_a, dst_a, av_a, sem_a)

        drain(src_b, dst_b, av_b, sem_b)
        accs = compute(src_b, dst_b, av_b, accs)

        @pl.when(2 * i + 3 < N_CHUNKS)
        def _():
            start(2 * i + 3, src_b, dst_b, av_b, sem_b)

        return accs

    accs = lax.fori_loop(0, N_CHUNKS // 2, outer,
                         (zero,) * UNROLL)
    a1 = accs[0] + accs[1] + accs[2] + accs[3] + accs[4]

    stage[pl.ds(0, L)] = a1
    stage[pl.ds(L, L)] = a2
    stage[pl.ds(2 * L, L)] = a3
    stage[pl.ds(3 * L, L)] = a4
    stage[pl.ds(4 * L, L)] = a5
    pltpu.sync_copy(stage, out_hbm.at[wid])


@jax.jit
def _run(d_pad, ei_flat, mv, r_pad, gt_pad):
    mesh = plsc.VectorSubcoreMesh(core_axis_name="c", subcore_axis_name="s")
    f = pl.kernel(
        _sc_body,
        out_type=jax.ShapeDtypeStruct((NW, 5 * L), jnp.float32),
        mesh=mesh,
        scratch_types=[
            pltpu.VMEM((PAD_N,), jnp.float32),
            pltpu.VMEM((CHUNK,), jnp.int32),
            pltpu.VMEM((CHUNK,), jnp.int32),
            pltpu.VMEM((CHUNK,), jnp.float32),
            pltpu.VMEM((CHUNK,), jnp.int32),
            pltpu.VMEM((CHUNK,), jnp.int32),
            pltpu.VMEM((CHUNK,), jnp.float32),
            pltpu.VMEM((5 * L,), jnp.float32),
            pltpu.SemaphoreType.DMA,
            pltpu.SemaphoreType.DMA,
        ],
        compiler_params=pltpu.CompilerParams(needs_layout_passes=False),
    )
    partials = f(d_pad, ei_flat, mv, r_pad, gt_pad)
    s = jnp.sum(partials.reshape(NW, 5, L), axis=(0, 2))
    alpha = s[1] / (s[0] + EPS)
    return (alpha * alpha * s[2] - 2.0 * alpha * s[3] + s[4]) / (s[4] + EPS)


def kernel(d, residual, gt, edge_index, matrix_values, mask, L_values,
           batch_ids):
    del mask, L_values, batch_ids
    pad = PAD_N - N_NODES
    d_pad = jnp.pad(d, (0, pad))
    r_pad = jnp.pad(residual, (0, pad))
    gt_pad = jnp.pad(gt, (0, pad))
    ei_flat = edge_index.astype(jnp.int32).reshape(-1)
    return _run(d_pad, ei_flat, matrix_values, r_pad, gt_pad)


# speedup vs baseline: 537.3563x; 1.0065x over previous
import jax
import jax.numpy as jnp
from jax import lax
from jax.experimental import pallas as pl
from jax.experimental.pallas import tpu as pltpu
from jax.experimental.pallas import tpu_sc as plsc

N_NODES = 100000
N_EDGES = 6400000
EPS = 1e-06

NW = 32
PAD_N = 102400
NODE_CHUNK = PAD_N // NW
EDGES_PER_TILE = N_EDGES // NW
CHUNK = 4000
N_CHUNKS = EDGES_PER_TILE // CHUNK
L = 16
UNROLL = 5


def _sc_body(d_hbm, ei_hbm, mv_hbm, r_hbm, gt_hbm, out_hbm,
             d_tab, src_a, dst_a, av_a, src_b, dst_b, av_b, stage,
             sem_a, sem_b):
    cid = lax.axis_index("c")
    sid = lax.axis_index("s")
    wid = sid * 2 + cid

    pltpu.sync_copy(d_hbm, d_tab)

    zero = jnp.zeros((L,), jnp.float32)

    nbase = wid * NODE_CHUNK
    pltpu.sync_copy(r_hbm.at[pl.ds(nbase, NODE_CHUNK)],
                    av_a.at[pl.ds(0, NODE_CHUNK)])
    pltpu.sync_copy(gt_hbm.at[pl.ds(nbase, NODE_CHUNK)],
                    av_b.at[pl.ds(0, NODE_CHUNK)])

    def node_step(i, accs):
        a2, a3, a4, a5 = accs
        for k in range(4):
            off = i * 4 * L + k * L
            dv = d_tab[pl.ds(nbase + off, L)]
            rv = av_a[pl.ds(off, L)]
            gv = av_b[pl.ds(off, L)]
            a2 = a2 + rv * dv
            a3 = a3 + dv * dv
            a4 = a4 + dv * gv
            a5 = a5 + gv * gv
        return (a2, a3, a4, a5)

    a2, a3, a4, a5 = lax.fori_loop(0, NODE_CHUNK // (4 * L), node_step,
                                   (zero, zero, zero, zero))

    ebase = wid * EDGES_PER_TILE

    def start(c, sbuf, dbuf, abuf, sem):
        base = ebase + c * CHUNK
        pltpu.async_copy(ei_hbm.at[pl.ds(base, CHUNK)], sbuf, sem)
        pltpu.async_copy(ei_hbm.at[pl.ds(N_EDGES + base, CHUNK)], dbuf, sem)
        pltpu.async_copy(mv_hbm.at[pl.ds(base, CHUNK)], abuf, sem)

    def drain(sbuf, dbuf, abuf, sem):
        dummy = ei_hbm.at[pl.ds(0, CHUNK)]
        pltpu.make_async_copy(dummy, sbuf, sem).wait()
        pltpu.make_async_copy(dummy, dbuf, sem).wait()
        pltpu.make_async_copy(dummy, abuf, sem).wait()

    def compute(sbuf, dbuf, abuf, accs):
        def step(j, accs_in):
            out = []
            for k in range(UNROLL):
                off = j * UNROLL * L + k * L
                si = sbuf[pl.ds(off, L)]
                ti = dbuf[pl.ds(off, L)]
                av = abuf[pl.ds(off, L)]
                dsv = plsc.load_gather(d_tab, [si])
                dtv = plsc.load_gather(d_tab, [ti])
                out.append(accs_in[k] + av * dsv * dtv)
            return tuple(out)
        return plsc.parallel_loop(0, CHUNK // (UNROLL * L),
                                  carry=accs, unroll=2)(step)

    start(0, src_a, dst_a, av_a, sem_a)
    start(1, src_b, dst_b, av_b, sem_b)

    def outer(i, accs):
        drain(src_a, dst_a, av_a, sem_a)
        accs = compute(src_a, dst_a, av_a, accs)

        @pl.when(2 * i + 2 < N_CHUNKS)
        def _():
            start(2 * i + 2, src_a, dst_a, av_a, sem_a)

        drain(src_b, dst_b, av_b, sem_b)
        accs = compute(src_b, dst_b, av_b, accs)

        @pl.when(2 * i + 3 < N_CHUNKS)
        def _():
            start(2 * i + 3, src_b, dst_b, av_b, sem_b)

        return accs

    accs = lax.fori_loop(0, N_CHUNKS // 2, outer,
                         (zero,) * UNROLL)
    a1 = accs[0] + accs[1] + accs[2] + accs[3] + accs[4]

    stage[pl.ds(0, L)] = a1
    stage[pl.ds(L, L)] = a2
    stage[pl.ds(2 * L, L)] = a3
    stage[pl.ds(3 * L, L)] = a4
    stage[pl.ds(4 * L, L)] = a5
    pltpu.sync_copy(stage, out_hbm.at[wid])


@jax.jit
def _run(d_pad, ei_flat, mv, r_pad, gt_pad):
    mesh = plsc.VectorSubcoreMesh(core_axis_name="c", subcore_axis_name="s")
    f = pl.kernel(
        _sc_body,
        out_type=jax.ShapeDtypeStruct((NW, 5 * L), jnp.float32),
        mesh=mesh,
        scratch_types=[
            pltpu.VMEM((PAD_N,), jnp.float32),
            pltpu.VMEM((CHUNK,), jnp.int32),
            pltpu.VMEM((CHUNK,), jnp.int32),
            pltpu.VMEM((CHUNK,), jnp.float32),
            pltpu.VMEM((CHUNK,), jnp.int32),
            pltpu.VMEM((CHUNK,), jnp.int32),
            pltpu.VMEM((CHUNK,), jnp.float32),
            pltpu.VMEM((5 * L,), jnp.float32),
            pltpu.SemaphoreType.DMA,
            pltpu.SemaphoreType.DMA,
        ],
        compiler_params=pltpu.CompilerParams(needs_layout_passes=False),
    )
    partials = f(d_pad, ei_flat, mv, r_pad, gt_pad)
    s = jnp.sum(partials.reshape(NW, 5, L), axis=(0, 2))
    alpha = s[1] / (s[0] + EPS)
    return (alpha * alpha * s[2] - 2.0 * alpha * s[3] + s[4]) / (s[4] + EPS)


def kernel(d, residual, gt, edge_index, matrix_values, mask, L_values,
           batch_ids):
    del mask, L_values, batch_ids
    pad = PAD_N - N_NODES
    d_pad = jnp.pad(d, (0, pad))
    r_pad = jnp.pad(residual, (0, pad))
    gt_pad = jnp.pad(gt, (0, pad))
    ei_flat = edge_index.astype(jnp.int32).reshape(-1)
    return _run(d_pad, ei_flat, matrix_values, r_pad, gt_pad)


# speedup vs baseline: 794.0603x; 1.4777x over previous
import jax
import jax.numpy as jnp
from jax import lax
from jax.experimental import pallas as pl
from jax.experimental.pallas import tpu as pltpu
from jax.experimental.pallas import tpu_sc as plsc

N_NODES = 100000
N_EDGES = 6400000
EPS = 1e-06

NW = 32
L = 16
CHUNK = 5120
N_CHUNKS = N_EDGES // CHUNK
OUTER = 20
UNROLL = 5
NODE_CHUNK = 4000
NODE_TILES = 25


def _sc_body(d_hbm, ei_hbm, mv_hbm, r_hbm, gt_hbm, out_hbm,
             d_tab, ei_a, av_a, ei_b, av_b, stage, sem_a, sem_b):
    cid = lax.axis_index("c")
    sid = lax.axis_index("s")
    wid = sid * 2 + cid

    pltpu.sync_copy(d_hbm, d_tab)

    zero = jnp.zeros((L,), jnp.float32)

    nbase = (wid % NODE_TILES) * NODE_CHUNK
    pltpu.sync_copy(r_hbm.at[pl.ds(nbase, NODE_CHUNK)],
                    av_a.at[pl.ds(0, NODE_CHUNK)])
    pltpu.sync_copy(gt_hbm.at[pl.ds(nbase, NODE_CHUNK)],
                    av_b.at[pl.ds(0, NODE_CHUNK)])

    def node_step(i, accs):
        a2, a3, a4, a5 = accs
        for k in range(UNROLL):
            off = i * UNROLL * L + k * L
            dv = d_tab[pl.ds(nbase + off, L)]
            rv = av_a[pl.ds(off, L)]
            gv = av_b[pl.ds(off, L)]
            a2 = a2 + rv * dv
            a3 = a3 + dv * dv
            a4 = a4 + dv * gv
            a5 = a5 + gv * gv
        return (a2, a3, a4, a5)

    a2, a3, a4, a5 = lax.fori_loop(0, NODE_CHUNK // (UNROLL * L), node_step,
                                   (zero, zero, zero, zero))
    node_on = wid < NODE_TILES
    a2 = jnp.where(node_on, a2, zero)
    a3 = jnp.where(node_on, a3, zero)
    a4 = jnp.where(node_on, a4, zero)
    a5 = jnp.where(node_on, a5, zero)


    def start(c, ei_buf, abuf, sem):
        base = c * CHUNK
        pltpu.async_copy(ei_hbm.at[:, pl.ds(base, CHUNK)], ei_buf, sem)
        pltpu.async_copy(mv_hbm.at[pl.ds(base, CHUNK)], abuf, sem)

    def drain(ei_buf, abuf, sem):
        pltpu.make_async_copy(ei_hbm.at[:, pl.ds(0, CHUNK)], ei_buf,
                              sem).wait()
        pltpu.make_async_copy(mv_hbm.at[pl.ds(0, CHUNK)], abuf, sem).wait()

    def compute(ei_buf, abuf, accs):
        def step(j, accs_in):
            out = []
            for k in range(UNROLL):
                off = j * UNROLL * L + k * L
                si = ei_buf[0, pl.ds(off, L)]
                ti = ei_buf[1, pl.ds(off, L)]
                av = abuf[pl.ds(off, L)]
                dsv = plsc.load_gather(d_tab, [si])
                dtv = plsc.load_gather(d_tab, [ti])
                out.append(accs_in[k] + av * dsv * dtv)
            return tuple(out)
        return plsc.parallel_loop(0, CHUNK // (UNROLL * L),
                                  carry=accs, unroll=2)(step)

    def masked(cond, new, old):
        return tuple(jnp.where(cond, n, o) for n, o in zip(new, old))

    start(wid, ei_a, av_a, sem_a)
    start(wid + NW, ei_b, av_b, sem_b)

    def outer(i, accs):
        ca = wid + 2 * NW * i
        on_a = ca < N_CHUNKS

        @pl.when(on_a)
        def _():
            drain(ei_a, av_a, sem_a)
        accs = masked(on_a, compute(ei_a, av_a, accs), accs)

        @pl.when(ca + 2 * NW < N_CHUNKS)
        def _():
            start(ca + 2 * NW, ei_a, av_a, sem_a)

        cb = ca + NW
        on_b = cb < N_CHUNKS

        @pl.when(on_b)
        def _():
            drain(ei_b, av_b, sem_b)
        accs = masked(on_b, compute(ei_b, av_b, accs), accs)

        @pl.when(cb + 2 * NW < N_CHUNKS)
        def _():
            start(cb + 2 * NW, ei_b, av_b, sem_b)

        return accs

    accs = lax.fori_loop(0, OUTER, outer, (zero,) * UNROLL)
    a1 = accs[0] + accs[1] + accs[2] + accs[3] + accs[4]

    stage[pl.ds(0, L)] = a1
    stage[pl.ds(L, L)] = a2
    stage[pl.ds(2 * L, L)] = a3
    stage[pl.ds(3 * L, L)] = a4
    stage[pl.ds(4 * L, L)] = a5
    pltpu.sync_copy(stage, out_hbm.at[wid])


@jax.jit
def _run(d, ei2, mv, r, gt):
    mesh = plsc.VectorSubcoreMesh(core_axis_name="c", subcore_axis_name="s")
    f = pl.kernel(
        _sc_body,
        out_type=jax.ShapeDtypeStruct((NW, 5 * L), jnp.float32),
        mesh=mesh,
        scratch_types=[
            pltpu.VMEM((N_NODES,), jnp.float32),
            pltpu.VMEM((2, CHUNK), jnp.int32),
            pltpu.VMEM((CHUNK,), jnp.float32),
            pltpu.VMEM((2, CHUNK), jnp.int32),
            pltpu.VMEM((CHUNK,), jnp.float32),
            pltpu.VMEM((5 * L,), jnp.float32),
            pltpu.SemaphoreType.DMA,
            pltpu.SemaphoreType.DMA,
        ],
        compiler_params=pltpu.CompilerParams(needs_layout_passes=False),
    )
    partials = f(d, ei2, mv, r, gt)
    s = jnp.sum(partials.reshape(NW, 5, L), axis=(0, 2))
    alpha = s[1] / (s[0] + EPS)
    return (alpha * alpha * s[2] - 2.0 * alpha * s[3] + s[4]) / (s[4] + EPS)


def kernel(d, residual, gt, edge_index, matrix_values, mask, L_values,
           batch_ids):
    del mask, L_values, batch_ids
    return _run(d, edge_index.astype(jnp.int32), matrix_values, residual, gt)
